# baseline (speedup 1.0000x reference)
import functools

import jax
import jax.numpy as jnp
from jax import lax
from jax.experimental import pallas as pl
from jax.experimental.pallas import tpu as pltpu
from jax.experimental.pallas import tpu_sc as plsc

N_EMB = 1000000
D_EMB = 32
BATCH = 16384

L = 16
NC = 2
NS = 16
NW = NC * NS
B_PER_W = BATCH // NW
CHUNK = 128
N_CHUNKS = B_PER_W // CHUNK
C_GROUPS = CHUNK // L

BLK = 8192
GRID = 123
Y_PAD = GRID * BLK
Y_ROWS = Y_PAD // 128


def _tc_body(tab_ref, w_ref, b_ref, y_ref):
    sums = jnp.sum(tab_ref[...] * w_ref[...], axis=1) + b_ref[0, 0]
    y_ref[...] = sums.reshape(BLK // 128, 128)


_tc_matvec = pl.pallas_call(
    _tc_body,
    grid=(GRID,),
    in_specs=[
        pl.BlockSpec((BLK, D_EMB), lambda i: (i, 0)),
        pl.BlockSpec((1, D_EMB), lambda i: (0, 0)),
        pl.BlockSpec((1, 1), lambda i: (0, 0)),
    ],
    out_specs=pl.BlockSpec((BLK // 128, 128), lambda i: (i, 0)),
    out_shape=jax.ShapeDtypeStruct((Y_ROWS, 128), jnp.float32),
)


@functools.partial(
    pl.kernel,
    mesh=plsc.VectorSubcoreMesh(core_axis_name="c", subcore_axis_name="s"),
    out_type=jax.ShapeDtypeStruct((BATCH,), jnp.float32),
    scratch_types=[
        pltpu.VMEM((B_PER_W,), jnp.int32),
        pltpu.VMEM((B_PER_W,), jnp.int32),
        pltpu.VMEM((CHUNK, 128), jnp.float32),
        pltpu.VMEM((B_PER_W,), jnp.float32),
        pltpu.SemaphoreType.DMA,
    ],
    compiler_params=pltpu.CompilerParams(needs_layout_passes=False),
)
def _sc_pick(idx_hbm, y2d_hbm, out_hbm, idx_v, rid_v, rows_v, out_v, sem):
    wid = lax.axis_index("s") * NC + lax.axis_index("c")
    base = wid * B_PER_W

    pltpu.sync_copy(idx_hbm.at[pl.ds(base, B_PER_W)], idx_v)

    def rid_body(t, carry):
        t0 = t * L
        rid_v[pl.ds(t0, L)] = lax.shift_right_logical(idx_v[pl.ds(t0, L)], 7)
        return carry

    lax.fori_loop(0, B_PER_W // L, rid_body, 0)

    lane = lax.iota(jnp.int32, L)

    def chunk_body(k, carry):
        k0 = k * CHUNK
        pltpu.async_copy(
            y2d_hbm.at[rid_v.at[pl.ds(k0, CHUNK)]], rows_v, sem).wait()
        for g in range(C_GROUPS):
            row0 = k0 + g * L
            sub = lax.bitwise_and(idx_v[pl.ds(row0, L)], 127)
            out_v[pl.ds(row0, L)] = plsc.load_gather(
                rows_v, [lane + g * L, sub])
        return carry

    lax.fori_loop(0, N_CHUNKS, chunk_body, 0)

    pltpu.sync_copy(out_v, out_hbm.at[pl.ds(base, B_PER_W)])


def kernel(idx, table, W, b):
    y2d = _tc_matvec(table, W.reshape(1, D_EMB), b.reshape(1, 1))
    y = _sc_pick(idx.astype(jnp.int32), y2d)
    return y.reshape(BATCH, 1)


# speedup vs baseline: 2.8196x; 2.8196x over previous
import functools

import jax
import jax.numpy as jnp
from jax import lax
from jax.experimental import pallas as pl
from jax.experimental.pallas import tpu as pltpu
from jax.experimental.pallas import tpu_sc as plsc

N_EMB = 1000000
D_EMB = 32
BATCH = 16384

L = 16
NC = 2
NS = 16
NW = NC * NS
B_PER_W = BATCH // NW
CHUNK = 128
N_CHUNKS = B_PER_W // CHUNK
GROUPS = B_PER_W // L


@functools.partial(
    pl.kernel,
    mesh=plsc.VectorSubcoreMesh(core_axis_name="c", subcore_axis_name="s"),
    out_type=jax.ShapeDtypeStruct((BATCH,), jnp.float32),
    scratch_types=[
        pltpu.VMEM((B_PER_W,), jnp.int32),
        pltpu.VMEM((B_PER_W, D_EMB), jnp.float32),
        pltpu.VMEM(((D_EMB + 1) * L,), jnp.float32),
        pltpu.VMEM((B_PER_W,), jnp.float32),
        pltpu.SemaphoreType.DMA,
    ],
    compiler_params=pltpu.CompilerParams(needs_layout_passes=False),
)
def _sc_gather_dot(idx_hbm, table_hbm, wb_hbm, out_hbm,
                   idx_v, rows_v, wb_v, out_v, sem):
    wid = lax.axis_index("s") * NC + lax.axis_index("c")
    base = wid * B_PER_W

    pltpu.sync_copy(idx_hbm.at[pl.ds(base, B_PER_W)], idx_v)
    pltpu.sync_copy(wb_hbm, wb_v)

    rows3 = rows_v.reshape(B_PER_W, 1, D_EMB)

    copies = []
    for j in range(N_CHUNKS):
        copies.append(pltpu.async_copy(
            table_hbm.at[idx_v.at[pl.ds(j * CHUNK, CHUNK)]],
            rows3.at[pl.ds(j * CHUNK, CHUNK)],
            sem,
        ))
    for c in copies:
        c.wait()

    ws = [wb_v[pl.ds(d * L, L)] for d in range(D_EMB)]
    bias = wb_v[pl.ds(D_EMB * L, L)]
    lane = lax.iota(jnp.int32, L)

    def body(g, carry):
        row0 = g * L
        rid = lane + row0
        acc = bias
        for d in range(D_EMB):
            col = plsc.load_gather(
                rows_v, [rid, jnp.full((L,), d, dtype=jnp.int32)])
            acc = acc + col * ws[d]
        out_v[pl.ds(row0, L)] = acc
        return carry

    lax.fori_loop(0, GROUPS, body, 0)

    pltpu.sync_copy(out_v, out_hbm.at[pl.ds(base, B_PER_W)])


def kernel(idx, table, W, b):
    table_lin = table.reshape(N_EMB, 1, D_EMB)
    wb = jnp.concatenate(
        [
            jnp.broadcast_to(W.reshape(D_EMB, 1), (D_EMB, L)),
            jnp.broadcast_to(b.reshape(1, 1), (1, L)),
        ],
        axis=0,
    ).reshape((D_EMB + 1) * L)
    y = _sc_gather_dot(idx.astype(jnp.int32), table_lin, wb)
    return y.reshape(BATCH, 1)
